# baseline (speedup 1.0000x reference)
import functools
import jax
import jax.numpy as jnp
from jax import lax
from jax.experimental import pallas as pl
from jax.experimental.pallas import tpu as pltpu, tpu_sc as plsc

_N = 2048
_BS = 8
_NSUB = 32
_QUART = _N // 4
_BIG = 3.0e38


def _sc_body(pts_hbm, gts_hbm, rowpart_hbm, colpart_hbm,
             px_v, py_v, pz_v, gx_v, gy_v, gz_v, rm_v, cm_v):
    cid = lax.axis_index("c")
    sid = lax.axis_index("s")
    wid = sid * 2 + cid
    b = wid // 4
    q = wid % 4
    base = q * _QUART

    pltpu.sync_copy(pts_hbm.at[pl.ds((b * 3 + 0) * _N + base, _QUART)], px_v)
    pltpu.sync_copy(pts_hbm.at[pl.ds((b * 3 + 1) * _N + base, _QUART)], py_v)
    pltpu.sync_copy(pts_hbm.at[pl.ds((b * 3 + 2) * _N + base, _QUART)], pz_v)
    pltpu.sync_copy(gts_hbm.at[pl.ds((b * 3 + 0) * _N, _N)], gx_v)
    pltpu.sync_copy(gts_hbm.at[pl.ds((b * 3 + 1) * _N, _N)], gy_v)
    pltpu.sync_copy(gts_hbm.at[pl.ds((b * 3 + 2) * _N, _N)], gz_v)

    def init_cm(j, carry):
        cm_v[pl.ds(j * 16, 16)] = jnp.full((16,), _BIG, jnp.float32)
        return carry

    lax.fori_loop(0, _N // 16, init_cm, 0)

    def outer(g16, carry):
        i0 = g16 * 16
        pxv = px_v[pl.ds(i0, 16)]
        pyv = py_v[pl.ds(i0, 16)]
        pzv = pz_v[pl.ds(i0, 16)]

        for half in range(2):
            ps = [(pxv[half * 8 + t], pyv[half * 8 + t], pzv[half * 8 + t])
                  for t in range(8)]

            def inner(j, rms, ps=ps):
                sl = pl.ds(j * 16, 16)
                gx = gx_v[sl]
                gy = gy_v[sl]
                gz = gz_v[sl]
                new_rms = []
                cmins = None
                for t in range(8):
                    dx = gx - ps[t][0]
                    dy = gy - ps[t][1]
                    dz = gz - ps[t][2]
                    d = dx * dx + dy * dy + dz * dz
                    new_rms.append(jnp.minimum(rms[t], d))
                    cmins = d if cmins is None else jnp.minimum(cmins, d)
                cm_v[sl] = jnp.minimum(cm_v[sl], cmins)
                return tuple(new_rms)

            big = jnp.full((16,), _BIG, jnp.float32)
            rms = lax.fori_loop(0, _N // 16, inner, (big,) * 8)
            for t in range(8):
                rm_v[pl.ds((i0 + half * 8 + t) * 16, 16)] = rms[t]
        return carry

    lax.fori_loop(0, _QUART // 16, outer, 0)

    pltpu.sync_copy(rm_v, rowpart_hbm.at[pl.ds(wid * _QUART * 16, _QUART * 16)])
    pltpu.sync_copy(cm_v, colpart_hbm.at[pl.ds(wid * _N, _N)])


def _sc_chamfer(pts_flat, gts_flat):
    mesh = plsc.VectorSubcoreMesh(core_axis_name="c", subcore_axis_name="s")
    f = functools.partial(
        pl.kernel,
        mesh=mesh,
        out_type=[
            jax.ShapeDtypeStruct((_NSUB * _QUART * 16,), jnp.float32),
            jax.ShapeDtypeStruct((_NSUB * _N,), jnp.float32),
        ],
        scratch_types=[
            pltpu.VMEM((_QUART,), jnp.float32),
            pltpu.VMEM((_QUART,), jnp.float32),
            pltpu.VMEM((_QUART,), jnp.float32),
            pltpu.VMEM((_N,), jnp.float32),
            pltpu.VMEM((_N,), jnp.float32),
            pltpu.VMEM((_N,), jnp.float32),
            pltpu.VMEM((_QUART * 16,), jnp.float32),
            pltpu.VMEM((_N,), jnp.float32),
        ],
    )(_sc_body)
    return f(pts_flat, gts_flat)


def kernel(points, gts):
    bs, n, _ = points.shape
    pts_flat = jnp.transpose(points, (0, 2, 1)).reshape(-1)
    gts_flat = jnp.transpose(gts, (0, 2, 1)).reshape(-1)
    rowpart, colpart = _sc_chamfer(pts_flat, gts_flat)
    rowmin = jnp.min(rowpart.reshape(bs, n, 16), axis=2)
    colmin = jnp.min(colpart.reshape(bs, 4, n), axis=1)
    p2g_b = jnp.sqrt(jnp.mean(rowmin, axis=1))
    g2p_b = jnp.sqrt(jnp.mean(colmin, axis=1))
    p2g = jnp.mean(p2g_b)
    g2p = jnp.mean(g2p_b)
    loss = (p2g + g2p) / 2.0
    return (loss, p2g, g2p)


# speedup vs baseline: 2.2397x; 2.2397x over previous
import functools
import jax
import jax.numpy as jnp
from jax import lax
from jax.experimental import pallas as pl
from jax.experimental.pallas import tpu as pltpu, tpu_sc as plsc

_N = 2048
_CHUNK = 512
_BIG = 3.0e38

_TC_BS = 6
_SC_BS = 2
_NSUB = 32
_SPB = _NSUB // _SC_BS
_ROWS = _N // _SPB


def _tc_body(p_ref, g_ref, p2g_ref, g2p_ref):
    pts = p_ref[0]
    g = g_ref[0]
    m = g.shape[1]
    px = pts[:, 0:1]
    py = pts[:, 1:2]
    pz = pts[:, 2:3]
    rowmin = None
    g2p_sum = None
    for k in range(0, m, _CHUNK):
        gx = g[0:1, k:k + _CHUNK]
        gy = g[1:2, k:k + _CHUNK]
        gz = g[2:3, k:k + _CHUNK]
        dx = px - gx
        dy = py - gy
        dz = pz - gz
        d = dx * dx + dy * dy + dz * dz
        rm = jnp.min(d, axis=1, keepdims=True)
        rowmin = rm if rowmin is None else jnp.minimum(rowmin, rm)
        cs = jnp.sum(jnp.min(d, axis=0))
        g2p_sum = cs if g2p_sum is None else g2p_sum + cs
    p2g_ref[0] = jnp.sqrt(jnp.mean(rowmin)).reshape(1, 1)
    g2p_ref[0] = jnp.sqrt(g2p_sum / m).reshape(1, 1)


def _tc_chamfer(points, gts_t):
    bs, n, _ = points.shape
    m = gts_t.shape[2]
    return pl.pallas_call(
        _tc_body,
        grid=(bs,),
        in_specs=[
            pl.BlockSpec((1, n, 3), lambda b: (b, 0, 0)),
            pl.BlockSpec((1, 3, m), lambda b: (b, 0, 0)),
        ],
        out_specs=[
            pl.BlockSpec((1, 1, 1), lambda b: (b, 0, 0)),
            pl.BlockSpec((1, 1, 1), lambda b: (b, 0, 0)),
        ],
        out_shape=[
            jax.ShapeDtypeStruct((bs, 1, 1), jnp.float32),
            jax.ShapeDtypeStruct((bs, 1, 1), jnp.float32),
        ],
    )(points, gts_t)


def _sc_body(pts_hbm, gts_hbm, rowpart_hbm, colpart_hbm,
             px_v, py_v, pz_v, gx_v, gy_v, gz_v, rm_v, cm_v):
    cid = lax.axis_index("c")
    sid = lax.axis_index("s")
    wid = sid * 2 + cid
    b = wid // _SPB
    base = (wid % _SPB) * _ROWS

    pltpu.sync_copy(pts_hbm.at[pl.ds((b * 3 + 0) * _N + base, _ROWS)], px_v)
    pltpu.sync_copy(pts_hbm.at[pl.ds((b * 3 + 1) * _N + base, _ROWS)], py_v)
    pltpu.sync_copy(pts_hbm.at[pl.ds((b * 3 + 2) * _N + base, _ROWS)], pz_v)
    pltpu.sync_copy(gts_hbm.at[pl.ds((b * 3 + 0) * _N, _N)], gx_v)
    pltpu.sync_copy(gts_hbm.at[pl.ds((b * 3 + 1) * _N, _N)], gy_v)
    pltpu.sync_copy(gts_hbm.at[pl.ds((b * 3 + 2) * _N, _N)], gz_v)

    def init_cm(j, carry):
        cm_v[pl.ds(j * 16, 16)] = jnp.full((16,), _BIG, jnp.float32)
        return carry

    lax.fori_loop(0, _N // 16, init_cm, 0)

    def outer(g16, carry):
        i0 = g16 * 16
        pxv = px_v[pl.ds(i0, 16)]
        pyv = py_v[pl.ds(i0, 16)]
        pzv = pz_v[pl.ds(i0, 16)]

        for half in range(2):
            ps = [(pxv[half * 8 + t], pyv[half * 8 + t], pzv[half * 8 + t])
                  for t in range(8)]

            def inner(j, rms, ps=ps):
                sl = pl.ds(j * 16, 16)
                gx = gx_v[sl]
                gy = gy_v[sl]
                gz = gz_v[sl]
                new_rms = []
                cmins = None
                for t in range(8):
                    dx = gx - ps[t][0]
                    dy = gy - ps[t][1]
                    dz = gz - ps[t][2]
                    d = dx * dx + dy * dy + dz * dz
                    new_rms.append(jnp.minimum(rms[t], d))
                    cmins = d if cmins is None else jnp.minimum(cmins, d)
                cm_v[sl] = jnp.minimum(cm_v[sl], cmins)
                return tuple(new_rms)

            big = jnp.full((16,), _BIG, jnp.float32)
            rms = lax.fori_loop(0, _N // 16, inner, (big,) * 8)
            for t in range(8):
                rm_v[pl.ds((i0 + half * 8 + t) * 16, 16)] = rms[t]
        return carry

    lax.fori_loop(0, _ROWS // 16, outer, 0)

    pltpu.sync_copy(rm_v, rowpart_hbm.at[pl.ds(wid * _ROWS * 16, _ROWS * 16)])
    pltpu.sync_copy(cm_v, colpart_hbm.at[pl.ds(wid * _N, _N)])


def _sc_chamfer(pts_flat, gts_flat):
    mesh = plsc.VectorSubcoreMesh(core_axis_name="c", subcore_axis_name="s")
    f = functools.partial(
        pl.kernel,
        mesh=mesh,
        out_type=[
            jax.ShapeDtypeStruct((_NSUB * _ROWS * 16,), jnp.float32),
            jax.ShapeDtypeStruct((_NSUB * _N,), jnp.float32),
        ],
        scratch_types=[
            pltpu.VMEM((_ROWS,), jnp.float32),
            pltpu.VMEM((_ROWS,), jnp.float32),
            pltpu.VMEM((_ROWS,), jnp.float32),
            pltpu.VMEM((_N,), jnp.float32),
            pltpu.VMEM((_N,), jnp.float32),
            pltpu.VMEM((_N,), jnp.float32),
            pltpu.VMEM((_ROWS * 16,), jnp.float32),
            pltpu.VMEM((_N,), jnp.float32),
        ],
    )(_sc_body)
    return f(pts_flat, gts_flat)


def kernel(points, gts):
    bs, n, _ = points.shape
    sc_pts = jnp.transpose(points[_TC_BS:], (0, 2, 1)).reshape(-1)
    sc_gts = jnp.transpose(gts[_TC_BS:], (0, 2, 1)).reshape(-1)
    rowpart, colpart = _sc_chamfer(sc_pts, sc_gts)

    tc_gts_t = jnp.transpose(gts[:_TC_BS], (0, 2, 1))
    p2g_tc, g2p_tc = _tc_chamfer(points[:_TC_BS], tc_gts_t)

    rowmin = jnp.min(rowpart.reshape(_SC_BS, n, 16), axis=2)
    colmin = jnp.min(colpart.reshape(_SC_BS, _SPB, n), axis=1)
    p2g_sc = jnp.sqrt(jnp.mean(rowmin, axis=1))
    g2p_sc = jnp.sqrt(jnp.mean(colmin, axis=1))

    p2g_all = jnp.concatenate([p2g_tc.reshape(-1), p2g_sc])
    g2p_all = jnp.concatenate([g2p_tc.reshape(-1), g2p_sc])
    p2g = jnp.mean(p2g_all)
    g2p = jnp.mean(g2p_all)
    loss = (p2g + g2p) / 2.0
    return (loss, p2g, g2p)


# speedup vs baseline: 2.4304x; 1.0851x over previous
import jax
import jax.numpy as jnp
from jax.experimental import pallas as pl

_CHUNK = 512


def _chamfer_body(p_ref, g_ref, p2g_ref, g2p_ref):
    pts = p_ref[0]
    gpts = g_ref[0]
    m = gpts.shape[0]
    px = pts[:, 0:1]
    py = pts[:, 1:2]
    pz = pts[:, 2:3]
    r = jax.lax.broadcasted_iota(jnp.int32, (3, 3), 0)
    c = jax.lax.broadcasted_iota(jnp.int32, (3, 3), 1)
    eye = jnp.where(r == c, 1.0, 0.0).astype(jnp.float32)
    g = jax.lax.dot_general(
        eye, gpts, (((1,), (1,)), ((), ())),
        precision=jax.lax.Precision.HIGHEST,
        preferred_element_type=jnp.float32,
    )
    rowmin = None
    g2p_sum = None
    for k in range(0, m, _CHUNK):
        gx = g[0:1, k:k + _CHUNK]
        gy = g[1:2, k:k + _CHUNK]
        gz = g[2:3, k:k + _CHUNK]
        dx = px - gx
        dy = py - gy
        dz = pz - gz
        d = dx * dx + dy * dy + dz * dz
        rm = jnp.min(d, axis=1, keepdims=True)
        rowmin = rm if rowmin is None else jnp.minimum(rowmin, rm)
        cs = jnp.sum(jnp.min(d, axis=0))
        g2p_sum = cs if g2p_sum is None else g2p_sum + cs
    p2g_ref[0] = jnp.sqrt(jnp.mean(rowmin)).reshape(1, 1)
    g2p_ref[0] = jnp.sqrt(g2p_sum / m).reshape(1, 1)


def kernel(points, gts):
    bs, n, _ = points.shape
    m = gts.shape[1]
    p2g_b, g2p_b = pl.pallas_call(
        _chamfer_body,
        grid=(bs,),
        in_specs=[
            pl.BlockSpec((1, n, 3), lambda b: (b, 0, 0)),
            pl.BlockSpec((1, m, 3), lambda b: (b, 0, 0)),
        ],
        out_specs=[
            pl.BlockSpec((1, 1, 1), lambda b: (b, 0, 0)),
            pl.BlockSpec((1, 1, 1), lambda b: (b, 0, 0)),
        ],
        out_shape=[
            jax.ShapeDtypeStruct((bs, 1, 1), jnp.float32),
            jax.ShapeDtypeStruct((bs, 1, 1), jnp.float32),
        ],
    )(points, gts)
    p2g = jnp.mean(p2g_b)
    g2p = jnp.mean(g2p_b)
    loss = (p2g + g2p) / 2.0
    return (loss, p2g, g2p)


# speedup vs baseline: 2.9288x; 1.2051x over previous
import jax
import jax.numpy as jnp
from jax.experimental import pallas as pl

_CHUNK = 512


def _chamfer_body(p_ref, g_ref, loss_ref, p2g_ref, g2p_ref):
    b = pl.program_id(0)
    bs = pl.num_programs(0)
    pts = p_ref[0]
    g = g_ref[0]
    m = g.shape[1]
    px = pts[:, 0:1]
    py = pts[:, 1:2]
    pz = pts[:, 2:3]
    rowmin = None
    g2p_sum = None
    for k in range(0, m, _CHUNK):
        gx = g[0:1, k:k + _CHUNK]
        gy = g[1:2, k:k + _CHUNK]
        gz = g[2:3, k:k + _CHUNK]
        dx = px - gx
        dy = py - gy
        dz = pz - gz
        d = dx * dx + dy * dy + dz * dz
        rm = jnp.min(d, axis=1, keepdims=True)
        rowmin = rm if rowmin is None else jnp.minimum(rowmin, rm)
        cs = jnp.sum(jnp.min(d, axis=0))
        g2p_sum = cs if g2p_sum is None else g2p_sum + cs
    p2g_b = jnp.sqrt(jnp.mean(rowmin)).reshape(1, 1) / bs
    g2p_b = jnp.sqrt(g2p_sum / m).reshape(1, 1) / bs

    @pl.when(b == 0)
    def _init():
        p2g_ref[0] = p2g_b
        g2p_ref[0] = g2p_b

    @pl.when(b > 0)
    def _acc():
        p2g_ref[0] += p2g_b
        g2p_ref[0] += g2p_b

    @pl.when(b == bs - 1)
    def _fin():
        loss_ref[0] = (p2g_ref[0] + g2p_ref[0]) * 0.5


def kernel(points, gts):
    bs, n, _ = points.shape
    m = gts.shape[1]
    gts_t = jnp.transpose(gts, (0, 2, 1))
    loss, p2g, g2p = pl.pallas_call(
        _chamfer_body,
        grid=(bs,),
        in_specs=[
            pl.BlockSpec((1, n, 3), lambda b: (b, 0, 0)),
            pl.BlockSpec((1, 3, m), lambda b: (b, 0, 0)),
        ],
        out_specs=[
            pl.BlockSpec((1, 1, 1), lambda b: (0, 0, 0)),
            pl.BlockSpec((1, 1, 1), lambda b: (0, 0, 0)),
            pl.BlockSpec((1, 1, 1), lambda b: (0, 0, 0)),
        ],
        out_shape=[
            jax.ShapeDtypeStruct((1, 1, 1), jnp.float32),
            jax.ShapeDtypeStruct((1, 1, 1), jnp.float32),
            jax.ShapeDtypeStruct((1, 1, 1), jnp.float32),
        ],
    )(points, gts_t)
    return (loss.reshape(()), p2g.reshape(()), g2p.reshape(()))
